# baseline (speedup 1.0000x reference)
import functools

import jax
import jax.numpy as jnp
from jax import lax
from jax.experimental import pallas as pl
from jax.experimental.pallas import tpu as pltpu
from jax.experimental.pallas import tpu_sc as plsc

_NC = 2
_NS = 16
_NW = _NC * _NS
_L = 16

_SC_ROWS = 256
_CHUNK_ROWS = 4
_TC_BLOCK_ROWS = 256


def _tc_body(x_ref, mins_ref, flag_ref, *, last_dims_equal):
    mins_ref[...] = jnp.min(x_ref[...], axis=1)

    @pl.when(pl.program_id(0) == 0)
    def _():
        flag_ref[...] = jnp.full((1, 1), 1.0 if last_dims_equal else 0.0,
                                 jnp.float32)


def _sc_body(x_hbm, mins_hbm, flag_hbm, buf, out_v, flag_v, sem0, sem1,
             *, row_offset, rows, cols, last_dims_equal):
    rows_per_w = rows // _NW
    nchunks = rows_per_w // _CHUNK_ROWS
    wid = lax.axis_index("s") * _NC + lax.axis_index("c")
    out_base = wid * rows_per_w
    base = row_offset + out_base
    sems = (sem0, sem1)

    def issue(c, slot):
        return pltpu.async_copy(
            x_hbm.at[pl.ds(base + c * _CHUNK_ROWS, _CHUNK_ROWS)],
            buf.at[slot], sems[slot])

    def lane_min_splat(v):
        for sh in (8, 4, 2, 1):
            idx = jnp.bitwise_xor(lax.iota(jnp.int32, _L), sh)
            v = jnp.minimum(v, jnp.take_along_axis(v, idx, axis=0))
        return v

    def reduce_chunk(c, slot):
        nacc = 4
        for r in range(_CHUNK_ROWS):
            def step(j, accs):
                return tuple(
                    jnp.minimum(accs[i],
                                buf[slot, r, pl.ds((j * nacc + i) * _L, _L)])
                    for i in range(nacc))
            accs = lax.fori_loop(
                0, cols // (nacc * _L), step,
                tuple(jnp.full((_L,), jnp.inf, jnp.float32)
                      for _ in range(nacc)),
                unroll=8)
            acc = jnp.minimum(jnp.minimum(accs[0], accs[1]),
                              jnp.minimum(accs[2], accs[3]))
            out_v[c * _CHUNK_ROWS + r, :] = lane_min_splat(acc)

    dsc = issue(0, 0)
    for c in range(nchunks):
        slot = c % 2
        dsc.wait()
        if c + 1 < nchunks:
            dsc = issue(c + 1, (c + 1) % 2)
        reduce_chunk(c, slot)

    pltpu.sync_copy(out_v, mins_hbm.at[pl.ds(out_base, rows_per_w)])

    @pl.when(wid == 0)
    def _():
        flag_v[...] = jnp.full((_L,), 1.0 if last_dims_equal else 0.0,
                               jnp.float32)
        pltpu.sync_copy(flag_v, flag_hbm)


def kernel(x):
    b0, b1, k = x.shape
    rows = b0 * b1
    xr = x.reshape(rows, k)
    last_dims_equal = b1 == k
    tc_rows = rows - _SC_ROWS

    sc_body = functools.partial(_sc_body, row_offset=tc_rows, rows=_SC_ROWS,
                                cols=k, last_dims_equal=last_dims_equal)
    sc_rows_per_w = _SC_ROWS // _NW
    mins_sc, flag_sc = pl.kernel(
        sc_body,
        out_type=[
            jax.ShapeDtypeStruct((_SC_ROWS, _L), jnp.float32),
            jax.ShapeDtypeStruct((_L,), jnp.float32),
        ],
        mesh=plsc.VectorSubcoreMesh(core_axis_name="c", subcore_axis_name="s"),
        scratch_types=[
            pltpu.VMEM((2, _CHUNK_ROWS, k), jnp.float32),
            pltpu.VMEM((sc_rows_per_w, _L), jnp.float32),
            pltpu.VMEM((_L,), jnp.float32),
            pltpu.SemaphoreType.DMA,
            pltpu.SemaphoreType.DMA,
        ],
    )(xr)

    tc_body = functools.partial(_tc_body, last_dims_equal=last_dims_equal)
    mins_tc, flag_tc = pl.pallas_call(
        tc_body,
        grid=(tc_rows // _TC_BLOCK_ROWS,),
        in_specs=[pl.BlockSpec((_TC_BLOCK_ROWS, k), lambda i: (i, 0))],
        out_specs=[
            pl.BlockSpec((_TC_BLOCK_ROWS,), lambda i: (i,)),
            pl.BlockSpec((1, 1), lambda i: (0, 0)),
        ],
        out_shape=[
            jax.ShapeDtypeStruct((tc_rows,), jnp.float32),
            jax.ShapeDtypeStruct((1, 1), jnp.float32),
        ],
    )(xr)

    del mins_tc, mins_sc
    return jnp.maximum(flag_tc[0, 0], flag_sc[0]).astype(jnp.bool_)


# speedup vs baseline: 2.4028x; 2.4028x over previous
import functools

import jax
import jax.numpy as jnp
from jax.experimental import pallas as pl
from jax.experimental.pallas import tpu as pltpu

_CH = 64
_NBUF = 6


def _body(x_hbm, mins_ref, flag_ref, buf, sems, *, rows, cols,
          last_dims_equal):
    nchunks = rows // _CH

    def copy(c, slot):
        return pltpu.make_async_copy(
            x_hbm.at[pl.ds(c * _CH, _CH)], buf.at[slot], sems.at[slot])

    for s in range(min(_NBUF, nchunks)):
        copy(s, s).start()
    for c in range(nchunks):
        slot = c % _NBUF
        copy(c, slot).wait()
        mins_ref[pl.ds(c * _CH, _CH)] = jnp.min(buf[slot], axis=1)
        nxt = c + _NBUF
        if nxt < nchunks:
            copy(nxt, slot).start()

    flag_ref[...] = jnp.full((1, 1), 1.0 if last_dims_equal else 0.0,
                             jnp.float32)


def kernel(x):
    b0, b1, k = x.shape
    rows = b0 * b1
    xr = x.reshape(rows, k)
    body = functools.partial(_body, rows=rows, cols=k,
                             last_dims_equal=(b1 == k))
    mins, flag = pl.pallas_call(
        body,
        in_specs=[pl.BlockSpec(memory_space=pl.ANY)],
        out_specs=[
            pl.BlockSpec(memory_space=pltpu.VMEM),
            pl.BlockSpec(memory_space=pltpu.VMEM),
        ],
        out_shape=[
            jax.ShapeDtypeStruct((rows,), jnp.float32),
            jax.ShapeDtypeStruct((1, 1), jnp.float32),
        ],
        scratch_shapes=[
            pltpu.VMEM((_NBUF, _CH, k), jnp.float32),
            pltpu.SemaphoreType.DMA((_NBUF,)),
        ],
    )(xr)
    del mins
    return flag[0, 0].astype(jnp.bool_)
